# baseline (speedup 1.0000x reference)
import functools

import jax
import jax.numpy as jnp
import numpy as np
from jax import lax
from jax.experimental import pallas as pl
from jax.experimental.pallas import tpu as pltpu
from jax.experimental.pallas import tpu_sc as plsc

_FEATURE_DIMS = [1000] * 26
_F = 26
_D = 16
_V = 26000
_B = 4096
_P = _F * (_F - 1) // 2
_IXW = _P * _D
_LINC = _IXW
_AUGW = 5376

_NC, _NS = 2, 16
_NW = _NC * _NS
_NB = _B // _NW
_CB = 4
_NCHUNK = _NB // _CB
_NIDX = _CB * _F

_BT = 256
_H = 400


def _sc_gather_interact(xo_hbm, tab_hbm, lin_hbm, feat_hbm,
                        idx_v, idx2_v, rows_v, lrows_v, ix_v, sem, sem2):
    w = lax.axis_index("s") * _NC + lax.axis_index("c")

    zero = jnp.zeros((_D,), jnp.float32)
    for bb in range(_CB):
        for col in range(_LINC + _D, _AUGW, _D):
            ix_v[bb, pl.ds(col, _D)] = zero

    lane = lax.iota(jnp.int32, _D)

    def chunk_body(c, carry):
        base_b = w * _NB + c * _CB
        pltpu.sync_copy(xo_hbm.at[pl.ds(base_b * _F, _NIDX)],
                        idx_v.at[pl.ds(0, _NIDX)])
        t = idx_v[pl.ds(96, _D)]
        idx_v[pl.ds(96, _D)] = jnp.where(lane < 8, t, 0)

        for bb in range(_CB):
            v0 = idx_v[pl.ds(bb * _F, _D)]
            v1 = idx_v[pl.ds(bb * _F + _D, _D)]
            for j in range(_F):
                idx2_v[j, pl.ds(bb * 32, _D)] = v0 + j * _V
                idx2_v[j, pl.ds(bb * 32 + _D, _D)] = v1 + j * _V

        ldesc = pltpu.async_copy(lin_hbm.at[idx_v], lrows_v, sem2)
        descs = [pltpu.async_copy(tab_hbm.at[idx2_v.at[j]], rows_v.at[j], sem)
                 for j in range(_F)]
        ldesc.wait()
        for d in descs:
            d.wait()

        for bb in range(_CB):
            lv = lrows_v[bb * _F, :]
            for i in range(1, _F):
                lv = lv + lrows_v[bb * _F + i, :]
            ix_v[bb, pl.ds(_LINC, _D)] = lv
            p = 0
            for i in range(_F):
                for j in range(i + 1, _F):
                    a = rows_v[j, bb * 32 + i, :]
                    b = rows_v[i, bb * 32 + j, :]
                    ix_v[bb, pl.ds(p * _D, _D)] = a * b
                    p += 1

        pltpu.sync_copy(ix_v, feat_hbm.at[pl.ds(base_b, _CB)])
        return carry

    lax.fori_loop(0, _NCHUNK, chunk_body, 0)


_sc_features = functools.partial(
    pl.kernel,
    out_type=jax.ShapeDtypeStruct((_B, _AUGW), jnp.float32),
    mesh=plsc.VectorSubcoreMesh(core_axis_name="c", subcore_axis_name="s"),
    compiler_params=pltpu.CompilerParams(use_tc_tiling_on_sc=False),
    scratch_types=[
        pltpu.VMEM((112,), jnp.int32),
        pltpu.VMEM((_F, 128), jnp.int32),
        pltpu.VMEM((_F, 128, _D), jnp.float32),
        pltpu.VMEM((112, _D), jnp.float32),
        pltpu.VMEM((_CB, _AUGW), jnp.float32),
        pltpu.SemaphoreType.DMA,
        pltpu.SemaphoreType.DMA,
    ],
)(_sc_gather_interact)


def _tc_mlp(feat_ref, w1_ref, b1_ref, w2_ref, b2_ref, w3_ref, b3_ref,
            el_ref, out_ref):
    x = feat_ref[...]
    h = jnp.dot(x, w1_ref[...], preferred_element_type=jnp.float32)
    h = jnp.maximum(h + b1_ref[...], 0.0)
    h = jnp.dot(h, w2_ref[...], preferred_element_type=jnp.float32)
    h = jnp.maximum(h + b2_ref[...], 0.0)
    d = jnp.dot(h, w3_ref[...], preferred_element_type=jnp.float32)
    lin = jnp.dot(x, el_ref[...], preferred_element_type=jnp.float32)
    z = d + lin + b3_ref[...]
    out_ref[...] = 1.0 / (1.0 + jnp.exp(-z))


def kernel(x, linear_w, linear_b, ffm_tables, W1, b1, W2, b2, W3, b3):
    offsets = jnp.asarray(
        np.concatenate([[0], np.cumsum(_FEATURE_DIMS)[:-1]]), dtype=x.dtype)
    xo = (x + offsets[None, :]).reshape(-1)

    tab = ffm_tables.reshape(_F * _V, _D)
    linpad = jnp.concatenate(
        [linear_w.reshape(_V, 1), jnp.zeros((_V, _D - 1), jnp.float32)],
        axis=1)

    feat = _sc_features(xo, tab, linpad)

    w1p = jnp.concatenate(
        [W1, jnp.zeros((_AUGW - _IXW, _H), jnp.float32)], axis=0)
    el = jnp.zeros((_AUGW, 1), jnp.float32).at[_LINC:_LINC + _D].set(1.0)
    b3c = (b3 + linear_b).reshape(1, 1)

    out2d = pl.pallas_call(
        _tc_mlp,
        grid=(_B // _BT,),
        in_specs=[
            pl.BlockSpec((_BT, _AUGW), lambda i: (i, 0)),
            pl.BlockSpec((_AUGW, _H), lambda i: (0, 0)),
            pl.BlockSpec((1, _H), lambda i: (0, 0)),
            pl.BlockSpec((_H, _H), lambda i: (0, 0)),
            pl.BlockSpec((1, _H), lambda i: (0, 0)),
            pl.BlockSpec((_H, 1), lambda i: (0, 0)),
            pl.BlockSpec((1, 1), lambda i: (0, 0)),
            pl.BlockSpec((_AUGW, 1), lambda i: (0, 0)),
        ],
        out_specs=pl.BlockSpec((_BT, 1), lambda i: (i, 0)),
        out_shape=jax.ShapeDtypeStruct((_B, 1), jnp.float32),
        compiler_params=pltpu.CompilerParams(
            dimension_semantics=("arbitrary",)),
    )(feat, w1p, b1.reshape(1, _H), W2, b2.reshape(1, _H), W3, b3c, el)

    return out2d.reshape(_B)


# speedup vs baseline: 1.0628x; 1.0628x over previous
import functools

import jax
import jax.numpy as jnp
import numpy as np
from jax import lax
from jax.experimental import pallas as pl
from jax.experimental.pallas import tpu as pltpu
from jax.experimental.pallas import tpu_sc as plsc

_FEATURE_DIMS = [1000] * 26
_F = 26
_D = 16
_V = 26000
_B = 4096
_P = _F * (_F - 1) // 2
_IXW = _P * _D
_LINC = _IXW
_NT = 42
_AUGW = _NT * 128
_LT, _LO = _LINC // 128, _LINC % 128

_NC, _NS = 2, 16
_NW = _NC * _NS
_NB = _B // _NW
_NSLAB = _NB // 8
_NIDX = _F * 4

_BT = 256
_H = 400


def _sc_gather_interact(xo_hbm, tab_hbm, lin_hbm, feat_hbm,
                        idx_v, idx2_v, rows_v, lrows_v, ix_v, sem, sem2):
    w = lax.axis_index("s") * _NC + lax.axis_index("c")

    zero = jnp.zeros((_D,), jnp.float32)
    for s in range(8):
        for o in range(_LO + _D, 128, _D):
            ix_v[_LT, s, pl.ds(o, _D)] = zero
        for o in range(0, 128, _D):
            ix_v[_NT - 1, s, pl.ds(o, _D)] = zero

    lane = lax.iota(jnp.int32, _D)

    def half_body(h, base_b):
        hb = base_b + h * 4
        pltpu.sync_copy(xo_hbm.at[pl.ds(hb * _F, _NIDX)],
                        idx_v.at[pl.ds(0, _NIDX)])
        t = idx_v[pl.ds(96, _D)]
        idx_v[pl.ds(96, _D)] = jnp.where(lane < 8, t, 0)

        for bb in range(4):
            v0 = idx_v[pl.ds(bb * _F, _D)]
            v1 = idx_v[pl.ds(bb * _F + _D, _D)]
            for j in range(_F):
                idx2_v[j, pl.ds(bb * 32, _D)] = v0 + j * _V
                idx2_v[j, pl.ds(bb * 32 + _D, _D)] = v1 + j * _V

        ldesc = pltpu.async_copy(lin_hbm.at[idx_v], lrows_v, sem2)
        descs = [pltpu.async_copy(tab_hbm.at[idx2_v.at[j]], rows_v.at[j], sem)
                 for j in range(_F)]
        ldesc.wait()
        for d in descs:
            d.wait()

        def batch_body(bb, sub):
            s = h * 4 + bb
            lv = lrows_v[bb * _F, :]
            for i in range(1, _F):
                lv = lv + lrows_v[bb * _F + i, :]
            ix_v[_LT, s, pl.ds(_LO, _D)] = lv
            p = 0
            for i in range(_F):
                for j in range(i + 1, _F):
                    a = rows_v[j, bb * 32 + i, :]
                    b = rows_v[i, bb * 32 + j, :]
                    ix_v[p // 8, s, pl.ds((p % 8) * _D, _D)] = a * b
                    p += 1
            return sub

        lax.fori_loop(0, 4, batch_body, 0)
        return base_b

    def slab_body(c, carry):
        base_b = w * _NB + c * 8
        lax.fori_loop(0, 2, half_body, base_b)
        pltpu.sync_copy(ix_v, feat_hbm.at[w * _NSLAB + c])
        return carry

    lax.fori_loop(0, _NSLAB, slab_body, 0)


_sc_features = functools.partial(
    pl.kernel,
    out_type=jax.ShapeDtypeStruct((_B // 8, _NT, 8, 128), jnp.float32),
    mesh=plsc.VectorSubcoreMesh(core_axis_name="c", subcore_axis_name="s"),
    compiler_params=pltpu.CompilerParams(use_tc_tiling_on_sc=False),
    scratch_types=[
        pltpu.VMEM((112,), jnp.int32),
        pltpu.VMEM((_F, 128), jnp.int32),
        pltpu.VMEM((_F, 128, _D), jnp.float32),
        pltpu.VMEM((112, _D), jnp.float32),
        pltpu.VMEM((_NT, 8, 128), jnp.float32),
        pltpu.SemaphoreType.DMA,
        pltpu.SemaphoreType.DMA,
    ],
)(_sc_gather_interact)


def _tc_mlp(feat_ref, w1_ref, b1_ref, w2_ref, b2_ref, w3_ref, b3_ref,
            el_ref, out_ref):
    h = jnp.dot(feat_ref[:, 0].reshape(_BT, 128), w1_ref[0],
                preferred_element_type=jnp.float32)
    for t in range(1, _NT):
        h = h + jnp.dot(feat_ref[:, t].reshape(_BT, 128), w1_ref[t],
                        preferred_element_type=jnp.float32)
    h = jnp.maximum(h + b1_ref[...], 0.0)
    h = jnp.dot(h, w2_ref[...], preferred_element_type=jnp.float32)
    h = jnp.maximum(h + b2_ref[...], 0.0)
    d = jnp.dot(h, w3_ref[...], preferred_element_type=jnp.float32)
    lin = jnp.dot(feat_ref[:, _LT].reshape(_BT, 128), el_ref[...],
                  preferred_element_type=jnp.float32)
    z = d + lin + b3_ref[...]
    out_ref[...] = 1.0 / (1.0 + jnp.exp(-z))


def kernel(x, linear_w, linear_b, ffm_tables, W1, b1, W2, b2, W3, b3):
    offsets = jnp.asarray(
        np.concatenate([[0], np.cumsum(_FEATURE_DIMS)[:-1]]), dtype=x.dtype)
    xo = (x + offsets[None, :]).reshape(-1)

    tab = ffm_tables.reshape(_F * _V, _D)
    linpad = jnp.concatenate(
        [linear_w.reshape(_V, 1), jnp.zeros((_V, _D - 1), jnp.float32)],
        axis=1)

    feat4 = _sc_features(xo, tab, linpad)

    w1r = jnp.concatenate(
        [W1, jnp.zeros((_AUGW - _IXW, _H), jnp.float32)],
        axis=0).reshape(_NT, 128, _H)
    el = jnp.zeros((128, 1), jnp.float32).at[_LO:_LO + _D].set(1.0)
    b3c = (b3 + linear_b).reshape(1, 1)

    out2d = pl.pallas_call(
        _tc_mlp,
        grid=(_B // _BT,),
        in_specs=[
            pl.BlockSpec((_BT // 8, _NT, 8, 128), lambda i: (i, 0, 0, 0)),
            pl.BlockSpec((_NT, 128, _H), lambda i: (0, 0, 0)),
            pl.BlockSpec((1, _H), lambda i: (0, 0)),
            pl.BlockSpec((_H, _H), lambda i: (0, 0)),
            pl.BlockSpec((1, _H), lambda i: (0, 0)),
            pl.BlockSpec((_H, 1), lambda i: (0, 0)),
            pl.BlockSpec((1, 1), lambda i: (0, 0)),
            pl.BlockSpec((128, 1), lambda i: (0, 0)),
        ],
        out_specs=pl.BlockSpec((_BT, 1), lambda i: (i, 0)),
        out_shape=jax.ShapeDtypeStruct((_B, 1), jnp.float32),
        compiler_params=pltpu.CompilerParams(
            dimension_semantics=("arbitrary",)),
    )(feat4, w1r, b1.reshape(1, _H), W2, b2.reshape(1, _H), W3, b3c, el)

    return out2d.reshape(_B)


# speedup vs baseline: 1.1944x; 1.1239x over previous
import functools

import jax
import jax.numpy as jnp
import numpy as np
from jax import lax
from jax.experimental import pallas as pl
from jax.experimental.pallas import tpu as pltpu
from jax.experimental.pallas import tpu_sc as plsc

_FEATURE_DIMS = [1000] * 26
_F = 26
_D = 16
_V = 26000
_B = 4096
_P = _F * (_F - 1) // 2
_IXW = _P * _D
_LINC = _IXW
_AUGW = 5376
_TABW = 512

_NC, _NS = 2, 16
_NW = _NC * _NS
_NB = _B // _NW
_CB = 4
_NCHUNK = _NB // _CB
_ROWS = _CB * _F

_BT = 256
_H = 400


def _sc_gather_interact(xo_hbm, tab_hbm, feat_hbm, idx_v, rows_v, ix_v, sem):
    w = lax.axis_index("s") * _NC + lax.axis_index("c")

    zero = jnp.zeros((_D,), jnp.float32)
    for bb in range(_CB):
        for col in range(_LINC + _D, _AUGW, _D):
            ix_v[bb, pl.ds(col, _D)] = zero

    def chunk_body(c, carry):
        base_b = w * _NB + c * _CB
        pltpu.sync_copy(xo_hbm.at[pl.ds(base_b * _F, _ROWS)], idx_v)
        descs = [pltpu.async_copy(tab_hbm.at[idx_v.at[pl.ds(k * 8, 8)]],
                                  rows_v.at[pl.ds(k * 8, 8)], sem)
                 for k in range(_ROWS // 8)]
        waited = 0
        for bb in range(_CB):
            need = (_F * (bb + 1) + 7) // 8
            while waited < need:
                descs[waited].wait()
                waited += 1
            rbase = bb * _F
            lv = rows_v[rbase, pl.ds(_F * _D, _D)]
            for i in range(1, _F):
                lv = lv + rows_v[rbase + i, pl.ds(_F * _D, _D)]
            ix_v[bb, pl.ds(_LINC, _D)] = lv
            p = 0
            for i in range(_F):
                for j in range(i + 1, _F):
                    a = rows_v[rbase + i, pl.ds(j * _D, _D)]
                    b = rows_v[rbase + j, pl.ds(i * _D, _D)]
                    ix_v[bb, pl.ds(p * _D, _D)] = a * b
                    p += 1
        pltpu.sync_copy(ix_v, feat_hbm.at[pl.ds(base_b, _CB)])
        return carry

    lax.fori_loop(0, _NCHUNK, chunk_body, 0)


_sc_features = functools.partial(
    pl.kernel,
    out_type=jax.ShapeDtypeStruct((_B, _AUGW), jnp.float32),
    mesh=plsc.VectorSubcoreMesh(core_axis_name="c", subcore_axis_name="s"),
    scratch_types=[
        pltpu.VMEM((_ROWS,), jnp.int32),
        pltpu.VMEM((_ROWS, _TABW), jnp.float32),
        pltpu.VMEM((_CB, _AUGW), jnp.float32),
        pltpu.SemaphoreType.DMA,
    ],
)(_sc_gather_interact)


def _tc_mlp(feat_ref, w1_ref, b1_ref, w2_ref, b2_ref, w3_ref, b3_ref,
            el_ref, out_ref):
    x = feat_ref[...]
    h = jnp.dot(x, w1_ref[...], preferred_element_type=jnp.float32)
    h = jnp.maximum(h + b1_ref[...], 0.0)
    h = jnp.dot(h, w2_ref[...], preferred_element_type=jnp.float32)
    h = jnp.maximum(h + b2_ref[...], 0.0)
    d = jnp.dot(h, w3_ref[...], preferred_element_type=jnp.float32)
    lin = jnp.dot(x, el_ref[...], preferred_element_type=jnp.float32)
    z = d + lin + b3_ref[...]
    out_ref[...] = 1.0 / (1.0 + jnp.exp(-z))


def kernel(x, linear_w, linear_b, ffm_tables, W1, b1, W2, b2, W3, b3):
    offsets = jnp.asarray(
        np.concatenate([[0], np.cumsum(_FEATURE_DIMS)[:-1]]), dtype=x.dtype)
    xo = (x + offsets[None, :]).reshape(-1)

    tab = jnp.transpose(ffm_tables, (1, 0, 2)).reshape(_V, _F * _D)
    tab = jnp.concatenate(
        [tab, linear_w.reshape(_V, 1),
         jnp.zeros((_V, _TABW - _F * _D - 1), jnp.float32)], axis=1)

    feat = _sc_features(xo, tab)

    w1p = jnp.concatenate(
        [W1, jnp.zeros((_AUGW - _IXW, _H), jnp.float32)], axis=0)
    el = jnp.zeros((_AUGW, 1), jnp.float32).at[_LINC:_LINC + _D].set(1.0)
    b3c = (b3 + linear_b).reshape(1, 1)

    out2d = pl.pallas_call(
        _tc_mlp,
        grid=(_B // _BT,),
        in_specs=[
            pl.BlockSpec((_BT, _AUGW), lambda i: (i, 0)),
            pl.BlockSpec((_AUGW, _H), lambda i: (0, 0)),
            pl.BlockSpec((1, _H), lambda i: (0, 0)),
            pl.BlockSpec((_H, _H), lambda i: (0, 0)),
            pl.BlockSpec((1, _H), lambda i: (0, 0)),
            pl.BlockSpec((_H, 1), lambda i: (0, 0)),
            pl.BlockSpec((1, 1), lambda i: (0, 0)),
            pl.BlockSpec((_AUGW, 1), lambda i: (0, 0)),
        ],
        out_specs=pl.BlockSpec((_BT, 1), lambda i: (i, 0)),
        out_shape=jax.ShapeDtypeStruct((_B, 1), jnp.float32),
        compiler_params=pltpu.CompilerParams(
            dimension_semantics=("arbitrary",)),
    )(feat, w1p, b1.reshape(1, _H), W2, b2.reshape(1, _H), W3, b3c, el)

    return out2d.reshape(_B)
